# baseline (speedup 1.0000x reference)
import functools

import jax
import jax.numpy as jnp
from jax import lax
from jax.experimental import pallas as pl
from jax.experimental.pallas import tpu as pltpu
from jax.experimental.pallas import tpu_sc as plsc

D_MODEL = 256
MAX_LEN = 100
B_TOTAL = 16 * 128 * 50
GRID = 100
SUB = 8
LANE = 128
NC = 2
NS = 16
NW = NC * NS
BPW = B_TOTAL // NW
RB = 64
NOUT = BPW // (2 * RB)
GPB = RB // 16


def _prep_body(xs_ref, ys_ref, idx_ref):
    qx = (((xs_ref[0] + 50.0) / 100.0) * (MAX_LEN - 1)).astype(jnp.int32)
    qy = (((ys_ref[0] + 50.0) / 100.0) * (MAX_LEN - 1)).astype(jnp.int32)
    qx = jnp.clip(qx, 0, MAX_LEN - 1)
    qy = jnp.clip(qy, 0, MAX_LEN - 1)
    idx_ref[0] = qx * 256 + qy


def _prep(xs, ys):
    return pl.pallas_call(
        _prep_body,
        grid=(GRID,),
        in_specs=[
            pl.BlockSpec((1, SUB, LANE), lambda i: (i, 0, 0)),
            pl.BlockSpec((1, SUB, LANE), lambda i: (i, 0, 0)),
        ],
        out_specs=pl.BlockSpec((1, SUB, LANE), lambda i: (i, 0, 0)),
        out_shape=jax.ShapeDtypeStruct((GRID, SUB, LANE), jnp.int32),
    )(xs, ys)


@functools.partial(
    pl.kernel,
    mesh=plsc.VectorSubcoreMesh(core_axis_name="c", subcore_axis_name="s"),
    compiler_params=pltpu.CompilerParams(needs_layout_passes=False),
    out_type=jax.ShapeDtypeStruct((B_TOTAL * D_MODEL,), jnp.float32),
    scratch_types=[
        pltpu.VMEM((MAX_LEN, D_MODEL), jnp.float32),
        pltpu.VMEM((BPW,), jnp.int32),
        pltpu.VMEM((RB * D_MODEL,), jnp.float32),
        pltpu.VMEM((RB * D_MODEL,), jnp.float32),
        pltpu.SemaphoreType.DMA,
        pltpu.SemaphoreType.DMA,
    ],
)
def _sc_embed(pe_hbm, idx_hbm, out_hbm, pe_v, idx_v, ob0, ob1, sem0, sem1):
    wid = lax.axis_index("s") * NC + lax.axis_index("c")
    base = wid * BPW
    pltpu.sync_copy(pe_hbm, pe_v)
    pltpu.sync_copy(idx_hbm.at[wid], idx_v)
    lanes = lax.iota(jnp.int32, 16)

    def fill(buf, g0):
        for sub in range(GPB):
            idx16 = idx_v[pl.ds((g0 + sub) * 16, 16)]
            qx = lax.shift_right_logical(idx16, 8)
            qy = lax.bitwise_and(idx16, 255)
            rowbase = (lanes + sub * 16) * D_MODEL

            def colstep(j, carry):
                colv = jnp.full((16,), j * 8, jnp.int32)
                for _ in range(8):
                    vx = plsc.load_gather(pe_v, [qx, colv])
                    vy = plsc.load_gather(pe_v, [qy, colv])
                    plsc.store_scatter(buf, [rowbase + colv], vx + vy)
                    colv = colv + 1
                return carry

            lax.fori_loop(0, D_MODEL // 8, colstep, 0)

    obytes = RB * D_MODEL
    obase = base * D_MODEL

    def body(i, carry):
        @pl.when(i > 0)
        def _():
            pltpu.make_async_copy(ob0, out_hbm.at[pl.ds(obase, obytes)], sem0).wait()

        fill(ob0, i * 2 * GPB)
        pltpu.async_copy(ob0, out_hbm.at[pl.ds(obase + i * 2 * obytes, obytes)], sem0)

        @pl.when(i > 0)
        def _():
            pltpu.make_async_copy(ob1, out_hbm.at[pl.ds(obase, obytes)], sem1).wait()

        fill(ob1, (i * 2 + 1) * GPB)
        pltpu.async_copy(
            ob1, out_hbm.at[pl.ds(obase + (i * 2 + 1) * obytes, obytes)], sem1)
        return carry

    lax.fori_loop(0, NOUT, body, 0)
    pltpu.make_async_copy(ob0, out_hbm.at[pl.ds(obase, obytes)], sem0).wait()
    pltpu.make_async_copy(ob1, out_hbm.at[pl.ds(obase, obytes)], sem1).wait()


def kernel(coords, pe):
    flat = coords.reshape(B_TOTAL, 2)
    xs = flat[:, 0].reshape(GRID, SUB, LANE)
    ys = flat[:, 1].reshape(GRID, SUB, LANE)
    idx = _prep(xs, ys)
    out = _sc_embed(pe, idx.reshape(NW, BPW))
    return out.reshape(coords.shape[0], coords.shape[1], coords.shape[2], D_MODEL)


_ = None


# speedup vs baseline: 3.2027x; 3.2027x over previous
import functools

import jax
import jax.numpy as jnp
from jax import lax
from jax.experimental import pallas as pl
from jax.experimental.pallas import tpu as pltpu
from jax.experimental.pallas import tpu_sc as plsc

D_MODEL = 256
MAX_LEN = 100
B_TOTAL = 16 * 128 * 50
GRID = 100
SUB = 8
LANE = 128
NC = 2
NS = 16
NW = NC * NS
BPW = B_TOTAL // NW
RB = 64
NOUT = BPW // (2 * RB)
GPB = RB // 16


def _prep_body(xs_ref, ys_ref, idx_ref):
    qx = (((xs_ref[0] + 50.0) / 100.0) * (MAX_LEN - 1)).astype(jnp.int32)
    qy = (((ys_ref[0] + 50.0) / 100.0) * (MAX_LEN - 1)).astype(jnp.int32)
    qx = jnp.clip(qx, 0, MAX_LEN - 1)
    qy = jnp.clip(qy, 0, MAX_LEN - 1)
    idx_ref[0] = qx * 256 + qy


def _prep(xs, ys):
    return pl.pallas_call(
        _prep_body,
        grid=(GRID,),
        in_specs=[
            pl.BlockSpec((1, SUB, LANE), lambda i: (i, 0, 0)),
            pl.BlockSpec((1, SUB, LANE), lambda i: (i, 0, 0)),
        ],
        out_specs=pl.BlockSpec((1, SUB, LANE), lambda i: (i, 0, 0)),
        out_shape=jax.ShapeDtypeStruct((GRID, SUB, LANE), jnp.int32),
    )(xs, ys)


@functools.partial(
    pl.kernel,
    mesh=plsc.VectorSubcoreMesh(core_axis_name="c", subcore_axis_name="s"),
    compiler_params=pltpu.CompilerParams(needs_layout_passes=False),
    out_type=jax.ShapeDtypeStruct((B_TOTAL, D_MODEL), jnp.float32),
    scratch_types=[
        pltpu.VMEM((MAX_LEN, D_MODEL), jnp.float32),
        pltpu.VMEM((BPW,), jnp.int32),
        pltpu.VMEM((RB, D_MODEL), jnp.float32),
        pltpu.VMEM((RB, D_MODEL), jnp.float32),
        pltpu.SemaphoreType.DMA,
        pltpu.SemaphoreType.DMA,
    ],
)
def _sc_embed(pe_hbm, idx_hbm, out_hbm, pe_v, idx_v, ob0, ob1, sem0, sem1):
    wid = lax.axis_index("s") * NC + lax.axis_index("c")
    base = wid * BPW
    pltpu.sync_copy(pe_hbm, pe_v)
    pltpu.sync_copy(idx_hbm.at[wid], idx_v)

    def fill(buf, g0):
        def subbody(sub, carry):
            idx16 = idx_v[pl.ds((g0 + sub) * 16, 16)]
            for r in range(16):
                q = idx16[r]
                qx = lax.shift_right_logical(q, 8)
                qy = lax.bitwise_and(q, 255)
                row = sub * 16 + r
                for c0 in range(0, D_MODEL, 16):
                    vx = pe_v[qx, pl.ds(c0, 16)]
                    vy = pe_v[qy, pl.ds(c0, 16)]
                    buf[row, pl.ds(c0, 16)] = vx + vy
            return carry

        lax.fori_loop(0, GPB, subbody, 0)

    def body(i, carry):
        @pl.when(i > 0)
        def _():
            pltpu.make_async_copy(ob0, out_hbm.at[pl.ds(base, RB)], sem0).wait()

        fill(ob0, i * 2 * GPB)
        pltpu.async_copy(ob0, out_hbm.at[pl.ds(base + i * 2 * RB, RB)], sem0)

        @pl.when(i > 0)
        def _():
            pltpu.make_async_copy(ob1, out_hbm.at[pl.ds(base, RB)], sem1).wait()

        fill(ob1, (i * 2 + 1) * GPB)
        pltpu.async_copy(ob1, out_hbm.at[pl.ds(base + i * 2 * RB + RB, RB)], sem1)
        return carry

    lax.fori_loop(0, NOUT, body, 0)
    pltpu.make_async_copy(ob0, out_hbm.at[pl.ds(base, RB)], sem0).wait()
    pltpu.make_async_copy(ob1, out_hbm.at[pl.ds(base, RB)], sem1).wait()


def kernel(coords, pe):
    flat = coords.reshape(B_TOTAL, 2)
    xs = flat[:, 0].reshape(GRID, SUB, LANE)
    ys = flat[:, 1].reshape(GRID, SUB, LANE)
    idx = _prep(xs, ys)
    out = _sc_embed(pe, idx.reshape(NW, BPW))
    return out.reshape(coords.shape[0], coords.shape[1], coords.shape[2], D_MODEL)
